# baseline (speedup 1.0000x reference)
import jax
import jax.numpy as jnp
from jax.experimental import pallas as pl
from jax.experimental.pallas import tpu as pltpu
from jax.experimental.pallas import tpu_sc as plsc

_BBLK = 8


def _sc_half(ids, matrix, which):
    b, s = ids.shape
    n, d = matrix.shape
    nblocks = b // _BBLK
    indices = ids.reshape(nblocks, _BBLK, s)

    mesh = plsc.VectorSubcoreMesh(
        core_axis_name="core", subcore_axis_name="subcore"
    )

    @pl.kernel(
        out_type=jax.ShapeDtypeStruct((b, s, d), matrix.dtype),
        mesh=mesh,
        scratch_types=[pltpu.SemaphoreType.DMA],
    )
    def gather_kernel(x_hbm, i_hbm, o_hbm, gsem):
        core = jax.lax.axis_index("core")

        def body(i_vmem, o_vmem):
            copies = [
                pltpu.async_copy(
                    x_hbm.at[i_vmem.at[0, j]], o_vmem.at[j], gsem
                )
                for j in range(_BBLK)
            ]
            for c in copies:
                c.wait()

        @pl.when(core == which)
        def _():
            pltpu.emit_pipeline(
                body,
                grid=(nblocks,),
                in_specs=[
                    pl.BlockSpec((1, _BBLK, s), index_map=lambda i: (i, 0, 0))
                ],
                out_specs=[
                    pl.BlockSpec((_BBLK, s, d), index_map=lambda i: (i, 0, 0))
                ],
                core_axis_name="subcore",
                dimension_semantics=(pltpu.PARALLEL,),
                trace_scopes=False,
            )(i_hbm, o_hbm)

    return gather_kernel(matrix, indices)


def kernel(token_ids, matrix):
    b = token_ids.shape[0]
    half = b // 2
    ids = token_ids.astype(jnp.int32)
    o0 = _sc_half(ids[:half], matrix, 0)
    o1 = _sc_half(ids[half:], matrix, 1)
    return o0, o1


# speedup vs baseline: 1.1675x; 1.1675x over previous
import jax
import jax.numpy as jnp
from jax.experimental import pallas as pl
from jax.experimental.pallas import tpu as pltpu
from jax.experimental.pallas import tpu_sc as plsc

_BBLK = 8


def kernel(token_ids, matrix):
    b, s = token_ids.shape
    n, d = matrix.shape
    nblocks = b // _BBLK
    indices = token_ids.astype(jnp.int32).reshape(nblocks, _BBLK, s)

    mesh = plsc.VectorSubcoreMesh(
        core_axis_name="core", subcore_axis_name="subcore"
    )

    @pl.kernel(
        out_type=jax.ShapeDtypeStruct((b, s, d), matrix.dtype),
        mesh=mesh,
        scratch_types=[pltpu.SemaphoreType.DMA],
    )
    def gather_kernel(x_hbm, i_hbm, o_hbm, gsem):
        def body(i_vmem, o_vmem):
            copies = [
                pltpu.async_copy(
                    x_hbm.at[i_vmem.at[0, j]], o_vmem.at[j], gsem
                )
                for j in range(_BBLK)
            ]
            for c in copies:
                c.wait()

        pltpu.emit_pipeline(
            body,
            grid=(nblocks,),
            in_specs=[
                pl.BlockSpec((1, _BBLK, s), index_map=lambda i: (i, 0, 0))
            ],
            out_specs=[
                pl.BlockSpec((_BBLK, s, d), index_map=lambda i: (i, 0, 0))
            ],
            core_axis_name=("core", "subcore"),
            dimension_semantics=(pltpu.PARALLEL,),
            trace_scopes=False,
        )(i_hbm, o_hbm)

    return gather_kernel(matrix, indices)
